# baseline (speedup 1.0000x reference)
import math

import jax
import jax.numpy as jnp
from jax.experimental import pallas as pl

_H = 16


def _mm(a, w):
    return jax.lax.dot_general(
        a.astype(jnp.bfloat16), w.astype(jnp.bfloat16),
        (((1,), (1,)), ((), ())), preferred_element_type=jnp.float32
    )


def _level_offsets(n):
    sizes = []
    c = n
    while c > 1:
        sizes.append(c // 2)
        c //= 2
    offs = [0]
    for s in sizes[:-1]:
        offs.append(offs[-1] + s)
    return sizes, offs


def _pairavg(a, pc, d):
    ar = a.reshape(pc, 2, d)
    return 0.5 * (ar[:, 0, :] + ar[:, 1, :])


def _tree_step(children, avg_c, wq, wk, wv, scale, h, dh):
    pc, d = avg_c.shape
    q = _mm(avg_c, wq)
    k = _mm(children, wk)
    v = _mm(children, wv)
    kr = k.reshape(pc, 2, d)
    vr = v.reshape(pc, 2, d)
    s0 = jnp.sum((q * kr[:, 0, :]).reshape(pc, h, dh), axis=-1)
    s1 = jnp.sum((q * kr[:, 1, :]).reshape(pc, h, dh), axis=-1)
    w0 = jax.nn.sigmoid((s0 - s1) * scale)
    w0f = jnp.broadcast_to(w0[:, :, None], (pc, h, dh)).reshape(pc, d)
    return w0f * vr[:, 0, :] + (1.0 - w0f) * vr[:, 1, :]


def _kv_of_parents(parents, wo, bo, wkx, wvx):
    y = _mm(parents, wo) + bo
    return _mm(y, wkx), _mm(y, wvx)


def _l0_kernel(x_ref, wq, wk, wv, wo, bo, wkx, wvx,
               par_ref, avg_ref, kp_ref, vp_ref):
    xb = x_ref[0]
    n2, d = xb.shape
    h, dh = _H, d // _H
    scale = 1.0 / math.sqrt(dh)
    avg_c = _pairavg(xb, n2 // 2, d)
    parents = _tree_step(xb, avg_c, wq[...], wk[...], wv[...], scale, h, dh)
    par_ref[0] = parents
    avg_ref[0] = avg_c
    kp, vp = _kv_of_parents(parents, wo[...], bo[...], wkx[...], wvx[...])
    kp_ref[0] = kp.astype(jnp.bfloat16)
    vp_ref[0] = vp.astype(jnp.bfloat16)


def _l1_kernel(ch_ref, avgp_ref, wq, wk, wv, wo, bo, wkx, wvx,
               par_ref, avg_ref, kp_ref, vp_ref):
    ch = ch_ref[0]
    n2, d = ch.shape
    h, dh = _H, d // _H
    scale = 1.0 / math.sqrt(dh)
    avg_c = _pairavg(avgp_ref[0], n2 // 2, d)
    parents = _tree_step(ch, avg_c, wq[...], wk[...], wv[...], scale, h, dh)
    par_ref[0] = parents
    avg_ref[0] = avg_c
    kp, vp = _kv_of_parents(parents, wo[...], bo[...], wkx[...], wvx[...])
    kp_ref[0] = kp.astype(jnp.bfloat16)
    vp_ref[0] = vp.astype(jnp.bfloat16)


def _rest_kernel(ch_ref, avgp_ref, wq, wk, wv, wo, bo, wkx, wvx,
                 kp_ref, vp_ref):
    ch = ch_ref[0]
    rows, d = ch.shape
    h, dh = _H, d // _H
    scale = 1.0 / math.sqrt(dh)
    children = ch
    avg_prev = avgp_ref[0]
    roff = 0
    pc = rows // 2
    while pc >= 1:
        avg_c = _pairavg(avg_prev, pc, d)
        parents = _tree_step(children, avg_c, wq[...], wk[...], wv[...],
                             scale, h, dh)
        kp, vp = _kv_of_parents(parents, wo[...], bo[...], wkx[...], wvx[...])
        kp_ref[0, roff:roff + pc, :] = kp.astype(jnp.bfloat16)
        vp_ref[0, roff:roff + pc, :] = vp.astype(jnp.bfloat16)
        children = parents
        avg_prev = avg_c
        roff += pc
        pc //= 2
    kp_ref[0, roff:roff + 1, :] = jnp.zeros((1, d), jnp.bfloat16)
    vp_ref[0, roff:roff + 1, :] = jnp.zeros((1, d), jnp.bfloat16)


def _proj3_kernel(x_ref, wq_ref, wk_ref, wv_ref, q_ref, k_ref, v_ref):
    xb = x_ref[0]
    q_ref[0] = _mm(xb, wq_ref[...]).astype(jnp.bfloat16)
    k_ref[0] = _mm(xb, wk_ref[...]).astype(jnp.bfloat16)
    v_ref[0] = _mm(xb, wv_ref[...]).astype(jnp.bfloat16)


def _attn_kernel(q_ref, kl_ref, vl_ref, ks, vs, ktail_ref, vtail_ref,
                 wo_ref, bo_ref, o_ref):
    q = q_ref[0].astype(jnp.float32)
    c, d = q.shape
    h, dh = _H, d // _H
    scale = 1.0 / math.sqrt(dh)
    j = pl.program_id(1)
    c0 = j * c
    L = 11

    def pairswap(a, m):
        ar = a.reshape(m // 2, 2, d)
        return jnp.concatenate([ar[:, 1:2, :], ar[:, 0:1, :]], axis=1).reshape(m, d)

    def expand(a, m):
        rep = c // m
        if rep == 1:
            return a
        return jnp.broadcast_to(a.reshape(m, 1, d), (m, rep, d)).reshape(c, d)

    def level_rows(l, leaf_ref, lvl_refs, tail_ref):
        if l == 0:
            return pairswap(leaf_ref[0], c)
        m = c >> l
        if m >= 8:
            return expand(pairswap(lvl_refs[l - 1][0], m), m)
        loc_off = {6: 0, 7: 32, 8: 48, 9: 56, 10: 60}[l]
        ii = jax.lax.broadcasted_iota(jnp.int32, (c, 64), 0)
        cols = jax.lax.broadcasted_iota(jnp.int32, (c, 64), 1)
        t = (((c0 + ii) >> l) ^ 1) + loc_off
        sel = jnp.where(cols == t, 1.0, 0.0).astype(jnp.bfloat16)
        return jax.lax.dot_general(
            sel, tail_ref[0], (((1,), (0,)), ((), ())),
            preferred_element_type=jnp.float32)

    scores = []
    for l in range(L):
        ke = level_rows(l, kl_ref, ks, ktail_ref).astype(jnp.float32)
        scores.append(jnp.sum((q * ke).reshape(c, h, dh), axis=-1) * scale)
    mx = scores[0]
    for s in scores[1:]:
        mx = jnp.maximum(mx, s)
    exps = [jnp.exp(s - mx) for s in scores]
    tot = exps[0]
    for e in exps[1:]:
        tot = tot + e
    inv = 1.0 / tot
    acc = jnp.zeros((c, d), jnp.float32)
    for l in range(L):
        w = exps[l] * inv
        wf = jnp.broadcast_to(w[:, :, None], (c, h, dh)).reshape(c, d)
        ve = level_rows(l, vl_ref, vs, vtail_ref).astype(jnp.float32)
        acc = acc + wf * ve
    o_ref[0] = _mm(acc, wo_ref[...]) + bo_ref[...]


def kernel(x, Wq_y, Wk_y, Wv_y, Wo_y, bo_y, Wq_x, Wk_x, Wv_x, Wo_x, bo_x):
    b, n, d = x.shape
    f32 = jnp.float32
    bf16 = jnp.bfloat16
    Wq_y, Wk_y, Wv_y, Wo_y = (w.astype(bf16) for w in (Wq_y, Wk_y, Wv_y, Wo_y))
    Wq_x, Wk_x, Wv_x, Wo_x = (w.astype(bf16) for w in (Wq_x, Wk_x, Wv_x, Wo_x))
    bo_y2 = bo_y.reshape(1, d)
    bo_x2 = bo_x.reshape(1, d)

    def wspec(nargs):
        return [pl.BlockSpec((d, d), lambda *a: (0, 0))] * nargs

    bspec = pl.BlockSpec((1, d), lambda *a: (0, 0))

    cp0 = 256
    g0 = (n // 2) // cp0
    p0, a0, kp0, vp0 = pl.pallas_call(
        _l0_kernel,
        grid=(b, g0),
        in_specs=[pl.BlockSpec((1, 2 * cp0, d), lambda i, j: (i, j, 0))]
        + wspec(4)[:4] + [bspec] + wspec(2),
        out_specs=[pl.BlockSpec((1, cp0, d), lambda i, j: (i, j, 0))] * 4,
        out_shape=[jax.ShapeDtypeStruct((b, n // 2, d), f32)] * 2
        + [jax.ShapeDtypeStruct((b, n // 2, d), bf16)] * 2,
    )(x, Wq_y, Wk_y, Wv_y, Wo_y, bo_y2, Wk_x, Wv_x)

    g1 = (n // 4) // cp0
    p1, a1, kp1, vp1 = pl.pallas_call(
        _l1_kernel,
        grid=(b, g1),
        in_specs=[pl.BlockSpec((1, 2 * cp0, d), lambda i, j: (i, j, 0))] * 2
        + wspec(4) + [bspec] + wspec(2),
        out_specs=[pl.BlockSpec((1, cp0, d), lambda i, j: (i, j, 0))] * 4,
        out_shape=[jax.ShapeDtypeStruct((b, n // 4, d), f32)] * 2
        + [jax.ShapeDtypeStruct((b, n // 4, d), bf16)] * 2,
    )(p0, a0, Wq_y, Wk_y, Wv_y, Wo_y, bo_y2, Wk_x, Wv_x)

    r = n // 4
    kpr, vpr = pl.pallas_call(
        _rest_kernel,
        grid=(b,),
        in_specs=[pl.BlockSpec((1, r, d), lambda i: (i, 0, 0))] * 2
        + wspec(4) + [bspec] + wspec(2),
        out_specs=[pl.BlockSpec((1, r, d), lambda i: (i, 0, 0))] * 2,
        out_shape=[jax.ShapeDtypeStruct((b, r, d), bf16)] * 2,
    )(p1, a1, Wq_y, Wk_y, Wv_y, Wo_y, bo_y2, Wk_x, Wv_x)

    nblk = 4
    blk = n // nblk
    blk_spec = pl.BlockSpec((1, blk, d), lambda i, j: (i, j, 0))
    qx, kl, vl = pl.pallas_call(
        _proj3_kernel,
        grid=(b, nblk),
        in_specs=[blk_spec] + wspec(3),
        out_specs=[blk_spec] * 3,
        out_shape=[jax.ShapeDtypeStruct((b, n, d), bf16)] * 3,
    )(x, Wq_x, Wk_x, Wv_x)

    cblk = 256
    anblk = n // cblk
    ablk_spec = pl.BlockSpec((1, cblk, d), lambda i, j: (i, j, 0))
    lvl_specs = []
    lvl_args_k = []
    lvl_args_v = []
    rest_boff = {3: 0, 4: 16, 5: 48}
    for l in range(1, 6):
        m = cblk >> l
        if l == 1:
            src_k, src_v = kp0, vp0
            boff = 0
        elif l == 2:
            src_k, src_v = kp1, vp1
            boff = 0
        else:
            src_k, src_v = kpr, vpr
            boff = rest_boff[l]
        lvl_specs.append(
            pl.BlockSpec((1, m, d), lambda i, j, boff=boff: (i, boff + j, 0)))
        lvl_args_k.append(src_k)
        lvl_args_v.append(src_v)
    tail_spec = pl.BlockSpec((1, 64, d), lambda i, j: (i, 7, 0))

    out = pl.pallas_call(
        lambda qr, klr, vlr, k1, k2, k3, k4, k5, kt,
        v1, v2, v3, v4, v5, vt, wor, bor, orf: _attn_kernel(
            qr, klr, vlr, [k1, k2, k3, k4, k5], [v1, v2, v3, v4, v5],
            kt, vt, wor, bor, orf),
        grid=(b, anblk),
        in_specs=[ablk_spec, ablk_spec, ablk_spec] + lvl_specs + [tail_spec]
        + lvl_specs + [tail_spec] + wspec(1) + [bspec],
        out_specs=ablk_spec,
        out_shape=jax.ShapeDtypeStruct((b, n, d), f32),
    )(qx, kl, vl, *lvl_args_k, kpr, *lvl_args_v, vpr, Wo_x, bo_x2)
    return out


# speedup vs baseline: 1.8085x; 1.8085x over previous
import math

import jax
import jax.numpy as jnp
from jax.experimental import pallas as pl

_H = 16


def _mm(a, w):
    return jax.lax.dot_general(
        a, w, (((1,), (1,)), ((), ())), preferred_element_type=jnp.float32
    )


def _level_offsets(n):
    sizes = []
    c = n
    while c > 1:
        sizes.append(c // 2)
        c //= 2
    offs = [0]
    for s in sizes[:-1]:
        offs.append(offs[-1] + s)
    return sizes, offs


def _pairavg(a, pc, d):
    ar = a.reshape(pc, 2, d)
    return 0.5 * (ar[:, 0, :] + ar[:, 1, :])


def _tree_step(children, avg_c, wq, wk, wv, scale, h, dh):
    pc, d = avg_c.shape
    q = _mm(avg_c, wq)
    k = _mm(children, wk)
    v = _mm(children, wv)
    kr = k.reshape(pc, 2, d)
    vr = v.reshape(pc, 2, d)
    s0 = jnp.sum((q * kr[:, 0, :]).reshape(pc, h, dh), axis=-1)
    s1 = jnp.sum((q * kr[:, 1, :]).reshape(pc, h, dh), axis=-1)
    w0 = jax.nn.sigmoid((s0 - s1) * scale)
    w0f = jnp.broadcast_to(w0[:, :, None], (pc, h, dh)).reshape(pc, d)
    return w0f * vr[:, 0, :] + (1.0 - w0f) * vr[:, 1, :]


def _kv_of_parents(parents, wo, bo, wkx, wvx):
    y = _mm(parents, wo) + bo
    return _mm(y, wkx), _mm(y, wvx)


def _l0_kernel(x_ref, wq, wk, wv, wo, bo, wkx, wvx,
               par_ref, avg_ref, kp_ref, vp_ref):
    xb = x_ref[0]
    n2, d = xb.shape
    h, dh = _H, d // _H
    scale = 1.0 / math.sqrt(dh)
    avg_c = _pairavg(xb, n2 // 2, d)
    parents = _tree_step(xb, avg_c, wq[...], wk[...], wv[...], scale, h, dh)
    par_ref[0] = parents
    avg_ref[0] = avg_c
    kp, vp = _kv_of_parents(parents, wo[...], bo[...], wkx[...], wvx[...])
    kp_ref[0] = kp
    vp_ref[0] = vp


def _l1_kernel(ch_ref, avgp_ref, wq, wk, wv, wo, bo, wkx, wvx,
               par_ref, avg_ref, kp_ref, vp_ref):
    ch = ch_ref[0]
    n2, d = ch.shape
    h, dh = _H, d // _H
    scale = 1.0 / math.sqrt(dh)
    avg_c = _pairavg(avgp_ref[0], n2 // 2, d)
    parents = _tree_step(ch, avg_c, wq[...], wk[...], wv[...], scale, h, dh)
    par_ref[0] = parents
    avg_ref[0] = avg_c
    kp, vp = _kv_of_parents(parents, wo[...], bo[...], wkx[...], wvx[...])
    kp_ref[0] = kp
    vp_ref[0] = vp


def _rest_kernel(ch_ref, avgp_ref, wq, wk, wv, wo, bo, wkx, wvx,
                 kp_ref, vp_ref):
    ch = ch_ref[0]
    rows, d = ch.shape
    h, dh = _H, d // _H
    scale = 1.0 / math.sqrt(dh)
    children = ch
    avg_prev = avgp_ref[0]
    roff = 0
    pc = rows // 2
    while pc >= 1:
        avg_c = _pairavg(avg_prev, pc, d)
        parents = _tree_step(children, avg_c, wq[...], wk[...], wv[...],
                             scale, h, dh)
        kp, vp = _kv_of_parents(parents, wo[...], bo[...], wkx[...], wvx[...])
        kp_ref[0, roff:roff + pc, :] = kp
        vp_ref[0, roff:roff + pc, :] = vp
        children = parents
        avg_prev = avg_c
        roff += pc
        pc //= 2
    kp_ref[0, roff:roff + 1, :] = jnp.zeros((1, d), jnp.float32)
    vp_ref[0, roff:roff + 1, :] = jnp.zeros((1, d), jnp.float32)


def _proj3_kernel(x_ref, wq_ref, wk_ref, wv_ref, q_ref, k_ref, v_ref):
    xb = x_ref[0]
    q_ref[0] = _mm(xb, wq_ref[...])
    k_ref[0] = _mm(xb, wk_ref[...])
    v_ref[0] = _mm(xb, wv_ref[...])


def _attn_kernel(x_ref, wqx_ref, wkx_ref, wvx_ref, ks, vs, ktail_ref,
                 vtail_ref, wo_ref, bo_ref, o_ref):
    xc = x_ref[0]
    q = _mm(xc, wqx_ref[...])
    kleaf = _mm(xc, wkx_ref[...])
    vleaf = _mm(xc, wvx_ref[...])
    c, d = q.shape
    h, dh = _H, d // _H
    scale = 1.0 / math.sqrt(dh)
    j = pl.program_id(1)
    c0 = j * c
    L = 11
    f32 = jnp.float32

    def iot(shape, dim):
        return jax.lax.broadcasted_iota(jnp.int32, shape, dim)

    seg = jnp.where(iot((d, h), 0) // dh == iot((d, h), 1), 1.0, 0.0)
    exp_m = jnp.where(iot((h, d), 0) == iot((h, d), 1) // dh, 1.0, 0.0)

    def dot(a, b):
        return jax.lax.dot_general(a, b, (((1,), (0,)), ((), ())),
                                   preferred_element_type=f32)

    sels = []
    for l in range(L):
        if l == 0:
            m = c
            t = iot((c, m), 0) ^ 1
        elif (c >> l) >= 8:
            m = c >> l
            t = (iot((c, m), 0) >> l) ^ 1
        else:
            m = 64
            loc_off = {6: 0, 7: 32, 8: 48, 9: 56, 10: 60}[l]
            t = (((c0 + iot((c, m), 0)) >> l) ^ 1) + loc_off
        sels.append(jnp.where(iot((c, m), 1) == t, 1.0, 0.0))

    def level_rows(l, leaf_val, lvl_refs, tail_ref):
        if l == 0:
            src = leaf_val
        elif (c >> l) >= 8:
            src = lvl_refs[l - 1][0]
        else:
            src = tail_ref[0]
        return dot(sels[l], src)

    scores = []
    for l in range(L):
        ke = level_rows(l, kleaf, ks, ktail_ref)
        scores.append(dot(q * ke, seg) * scale)
    mx = scores[0]
    for s in scores[1:]:
        mx = jnp.maximum(mx, s)
    exps = [jnp.exp(s - mx) for s in scores]
    tot = exps[0]
    for e in exps[1:]:
        tot = tot + e
    inv = 1.0 / tot
    acc = jnp.zeros((c, d), f32)
    for l in range(L):
        wf = dot(exps[l] * inv, exp_m)
        ve = level_rows(l, vleaf, vs, vtail_ref)
        acc = acc + wf * ve
    o_ref[0] = _mm(acc, wo_ref[...]) + bo_ref[...]


def kernel(x, Wq_y, Wk_y, Wv_y, Wo_y, bo_y, Wq_x, Wk_x, Wv_x, Wo_x, bo_x):
    b, n, d = x.shape
    f32 = jnp.float32
    bo_y2 = bo_y.reshape(1, d)
    bo_x2 = bo_x.reshape(1, d)

    def wspec(nargs):
        return [pl.BlockSpec((d, d), lambda *a: (0, 0))] * nargs

    bspec = pl.BlockSpec((1, d), lambda *a: (0, 0))

    cp0 = 256
    g0 = (n // 2) // cp0
    p0, a0, kp0, vp0 = pl.pallas_call(
        _l0_kernel,
        grid=(b, g0),
        in_specs=[pl.BlockSpec((1, 2 * cp0, d), lambda i, j: (i, j, 0))]
        + wspec(4)[:4] + [bspec] + wspec(2),
        out_specs=[pl.BlockSpec((1, cp0, d), lambda i, j: (i, j, 0))] * 4,
        out_shape=[jax.ShapeDtypeStruct((b, n // 2, d), f32)] * 4,
    )(x, Wq_y, Wk_y, Wv_y, Wo_y, bo_y2, Wk_x, Wv_x)

    g1 = (n // 4) // cp0
    p1, a1, kp1, vp1 = pl.pallas_call(
        _l1_kernel,
        grid=(b, g1),
        in_specs=[pl.BlockSpec((1, 2 * cp0, d), lambda i, j: (i, j, 0))] * 2
        + wspec(4) + [bspec] + wspec(2),
        out_specs=[pl.BlockSpec((1, cp0, d), lambda i, j: (i, j, 0))] * 4,
        out_shape=[jax.ShapeDtypeStruct((b, n // 4, d), f32)] * 4,
    )(p0, a0, Wq_y, Wk_y, Wv_y, Wo_y, bo_y2, Wk_x, Wv_x)

    r = n // 4
    kpr, vpr = pl.pallas_call(
        _rest_kernel,
        grid=(b,),
        in_specs=[pl.BlockSpec((1, r, d), lambda i: (i, 0, 0))] * 2
        + wspec(4) + [bspec] + wspec(2),
        out_specs=[pl.BlockSpec((1, r, d), lambda i: (i, 0, 0))] * 2,
        out_shape=[jax.ShapeDtypeStruct((b, r, d), f32)] * 2,
    )(p1, a1, Wq_y, Wk_y, Wv_y, Wo_y, bo_y2, Wk_x, Wv_x)

    cblk = 256
    anblk = n // cblk
    ablk_spec = pl.BlockSpec((1, cblk, d), lambda i, j: (i, j, 0))
    lvl_specs = []
    lvl_args_k = []
    lvl_args_v = []
    rest_boff = {3: 0, 4: 16, 5: 48}
    for l in range(1, 6):
        m = cblk >> l
        if l == 1:
            src_k, src_v = kp0, vp0
            boff = 0
        elif l == 2:
            src_k, src_v = kp1, vp1
            boff = 0
        else:
            src_k, src_v = kpr, vpr
            boff = rest_boff[l]
        lvl_specs.append(
            pl.BlockSpec((1, m, d), lambda i, j, boff=boff: (i, boff + j, 0)))
        lvl_args_k.append(src_k)
        lvl_args_v.append(src_v)
    tail_spec = pl.BlockSpec((1, 64, d), lambda i, j: (i, 7, 0))

    out = pl.pallas_call(
        lambda xr, wq, wk, wv, k1, k2, k3, k4, k5, kt,
        v1, v2, v3, v4, v5, vt, wor, bor, orf: _attn_kernel(
            xr, wq, wk, wv, [k1, k2, k3, k4, k5], [v1, v2, v3, v4, v5],
            kt, vt, wor, bor, orf),
        grid=(b, anblk),
        in_specs=[ablk_spec] + wspec(3) + lvl_specs + [tail_spec]
        + lvl_specs + [tail_spec] + wspec(1) + [bspec],
        out_specs=ablk_spec,
        out_shape=jax.ShapeDtypeStruct((b, n, d), f32),
    )(x, Wq_x, Wk_x, Wv_x, *lvl_args_k, kpr, *lvl_args_v, vpr, Wo_x, bo_x2)
    return out


# speedup vs baseline: 2.0621x; 1.1402x over previous
import math

import jax
import jax.numpy as jnp
from jax.experimental import pallas as pl

_H = 16


def _mm(a, w):
    return jax.lax.dot_general(
        a, w, (((1,), (1,)), ((), ())), preferred_element_type=jnp.float32
    )


def _level_offsets(n):
    sizes = []
    c = n
    while c > 1:
        sizes.append(c // 2)
        c //= 2
    offs = [0]
    for s in sizes[:-1]:
        offs.append(offs[-1] + s)
    return sizes, offs


def _pairavg(a, pc, d):
    ar = a.reshape(pc, 2, d)
    return 0.5 * (ar[:, 0, :] + ar[:, 1, :])


def _tree_step(children, avg_c, wq, wk, wv, scale, h, dh):
    pc, d = avg_c.shape
    q = _mm(avg_c, wq)
    k = _mm(children, wk)
    v = _mm(children, wv)
    kr = k.reshape(pc, 2, d)
    vr = v.reshape(pc, 2, d)
    s0 = jnp.sum((q * kr[:, 0, :]).reshape(pc, h, dh), axis=-1)
    s1 = jnp.sum((q * kr[:, 1, :]).reshape(pc, h, dh), axis=-1)
    w0 = jax.nn.sigmoid((s0 - s1) * scale)
    w0f = jnp.broadcast_to(w0[:, :, None], (pc, h, dh)).reshape(pc, d)
    return w0f * vr[:, 0, :] + (1.0 - w0f) * vr[:, 1, :]


def _kv_of_parents(parents, wo, bo, wkx, wvx):
    y = _mm(parents, wo) + bo
    return _mm(y, wkx), _mm(y, wvx)


def _l0_kernel(x_ref, wq, wk, wv, wo, bo, wkx, wvx,
               par_ref, avg_ref, kp_ref, vp_ref):
    nb, n2, d = x_ref.shape
    xb = x_ref[...].reshape(nb * n2, d)
    h, dh = _H, d // _H
    scale = 1.0 / math.sqrt(dh)
    pc = nb * n2 // 2
    avg_c = _pairavg(xb, pc, d)
    parents = _tree_step(xb, avg_c, wq[...], wk[...], wv[...], scale, h, dh)
    par_ref[...] = parents.reshape(nb, n2 // 2, d)
    avg_ref[...] = avg_c.reshape(nb, n2 // 2, d)
    kp, vp = _kv_of_parents(parents, wo[...], bo[...], wkx[...], wvx[...])
    kp_ref[...] = kp.reshape(nb, n2 // 2, d)
    vp_ref[...] = vp.reshape(nb, n2 // 2, d)


def _l1_kernel(ch_ref, avgp_ref, wq, wk, wv, wo, bo, wkx, wvx,
               par_ref, avg_ref, kp_ref, vp_ref):
    nb, n2, d = ch_ref.shape
    ch = ch_ref[...].reshape(nb * n2, d)
    h, dh = _H, d // _H
    scale = 1.0 / math.sqrt(dh)
    pc = nb * n2 // 2
    avg_c = _pairavg(avgp_ref[...].reshape(nb * n2, d), pc, d)
    parents = _tree_step(ch, avg_c, wq[...], wk[...], wv[...], scale, h, dh)
    par_ref[...] = parents.reshape(nb, n2 // 2, d)
    avg_ref[...] = avg_c.reshape(nb, n2 // 2, d)
    kp, vp = _kv_of_parents(parents, wo[...], bo[...], wkx[...], wvx[...])
    kp_ref[...] = kp.reshape(nb, n2 // 2, d)
    vp_ref[...] = vp.reshape(nb, n2 // 2, d)


def _rest_kernel(ch_ref, avgp_ref, wq, wk, wv, wo, bo, wkx, wvx,
                 kp_ref, vp_ref):
    nb, rows, d = ch_ref.shape
    h, dh = _H, d // _H
    scale = 1.0 / math.sqrt(dh)
    children = ch_ref[...].reshape(nb * rows, d)
    avg_prev = avgp_ref[...].reshape(nb * rows, d)
    roff = 0
    pc = rows // 2
    while pc >= 1:
        avg_c = _pairavg(avg_prev, nb * pc, d)
        parents = _tree_step(children, avg_c, wq[...], wk[...], wv[...],
                             scale, h, dh)
        kp, vp = _kv_of_parents(parents, wo[...], bo[...], wkx[...], wvx[...])
        kp_ref[:, roff:roff + pc, :] = kp.reshape(nb, pc, d)
        vp_ref[:, roff:roff + pc, :] = vp.reshape(nb, pc, d)
        children = parents
        avg_prev = avg_c
        roff += pc
        pc //= 2
    kp_ref[:, roff:roff + 1, :] = jnp.zeros((nb, 1, d), jnp.float32)
    vp_ref[:, roff:roff + 1, :] = jnp.zeros((nb, 1, d), jnp.float32)


def _proj3_kernel(x_ref, wq_ref, wk_ref, wv_ref, q_ref, k_ref, v_ref):
    xb = x_ref[0]
    q_ref[0] = _mm(xb, wq_ref[...])
    k_ref[0] = _mm(xb, wk_ref[...])
    v_ref[0] = _mm(xb, wv_ref[...])


def _attn_kernel(x_ref, wqx_ref, wkx_ref, wvx_ref, ks, vs, ktail_ref,
                 vtail_ref, wo_ref, bo_ref, o_ref):
    xc = x_ref[0]
    q = _mm(xc, wqx_ref[...])
    kleaf = _mm(xc, wkx_ref[...])
    vleaf = _mm(xc, wvx_ref[...])
    c, d = q.shape
    h, dh = _H, d // _H
    scale = 1.0 / math.sqrt(dh)
    j = pl.program_id(1)
    c0 = j * c
    L = 11
    f32 = jnp.float32

    def iot(shape, dim):
        return jax.lax.broadcasted_iota(jnp.int32, shape, dim)

    seg = jnp.where(iot((d, h), 0) // dh == iot((d, h), 1), 1.0, 0.0)
    exp_m = jnp.where(iot((h, d), 0) == iot((h, d), 1) // dh, 1.0, 0.0)

    def dot(a, b):
        return jax.lax.dot_general(a, b, (((1,), (0,)), ((), ())),
                                   preferred_element_type=f32)

    sels = []
    for l in range(L):
        if l == 0:
            m = c
            t = iot((c, m), 0) ^ 1
        elif (c >> l) >= 8:
            m = c >> l
            t = (iot((c, m), 0) >> l) ^ 1
        else:
            m = 64
            loc_off = {6: 0, 7: 32, 8: 48, 9: 56, 10: 60}[l]
            t = (((c0 + iot((c, m), 0)) >> l) ^ 1) + loc_off
        sels.append(jnp.where(iot((c, m), 1) == t, 1.0, 0.0))

    def level_rows(l, leaf_val, lvl_refs, tail_ref):
        if l == 0:
            src = leaf_val
        elif (c >> l) >= 8:
            src = lvl_refs[l - 1][0]
        else:
            src = tail_ref[0]
        return dot(sels[l], src)

    scores = []
    for l in range(L):
        ke = level_rows(l, kleaf, ks, ktail_ref)
        scores.append(dot(q * ke, seg) * scale)
    mx = scores[0]
    for s in scores[1:]:
        mx = jnp.maximum(mx, s)
    exps = [jnp.exp(s - mx) for s in scores]
    tot = exps[0]
    for e in exps[1:]:
        tot = tot + e
    inv = 1.0 / tot
    acc = jnp.zeros((c, d), f32)
    for l in range(L):
        wf = dot(exps[l] * inv, exp_m)
        ve = level_rows(l, vleaf, vs, vtail_ref)
        acc = acc + wf * ve
    o_ref[0] = _mm(acc, wo_ref[...]) + bo_ref[...]


def kernel(x, Wq_y, Wk_y, Wv_y, Wo_y, bo_y, Wq_x, Wk_x, Wv_x, Wo_x, bo_x):
    b, n, d = x.shape
    f32 = jnp.float32
    bo_y2 = bo_y.reshape(1, d)
    bo_x2 = bo_x.reshape(1, d)

    def wspec(nargs):
        return [pl.BlockSpec((d, d), lambda *a: (0, 0))] * nargs

    bspec = pl.BlockSpec((1, d), lambda *a: (0, 0))

    cp0 = 256
    g0 = (n // 2) // cp0
    p0, a0, kp0, vp0 = pl.pallas_call(
        _l0_kernel,
        grid=(g0,),
        in_specs=[pl.BlockSpec((b, 2 * cp0, d), lambda j: (0, j, 0))]
        + wspec(4) + [bspec] + wspec(2),
        out_specs=[pl.BlockSpec((b, cp0, d), lambda j: (0, j, 0))] * 4,
        out_shape=[jax.ShapeDtypeStruct((b, n // 2, d), f32)] * 4,
    )(x, Wq_y, Wk_y, Wv_y, Wo_y, bo_y2, Wk_x, Wv_x)

    cp1 = 128
    g1 = (n // 4) // cp1
    p1, a1, kp1, vp1 = pl.pallas_call(
        _l1_kernel,
        grid=(g1,),
        in_specs=[pl.BlockSpec((b, 2 * cp1, d), lambda j: (0, j, 0))] * 2
        + wspec(4) + [bspec] + wspec(2),
        out_specs=[pl.BlockSpec((b, cp1, d), lambda j: (0, j, 0))] * 4,
        out_shape=[jax.ShapeDtypeStruct((b, n // 4, d), f32)] * 4,
    )(p0, a0, Wq_y, Wk_y, Wv_y, Wo_y, bo_y2, Wk_x, Wv_x)

    r = n // 4
    kpr, vpr = pl.pallas_call(
        _rest_kernel,
        grid=(1,),
        in_specs=[pl.BlockSpec((b, r, d), lambda j: (0, 0, 0))] * 2
        + wspec(4) + [bspec] + wspec(2),
        out_specs=[pl.BlockSpec((b, r, d), lambda j: (0, 0, 0))] * 2,
        out_shape=[jax.ShapeDtypeStruct((b, r, d), f32)] * 2,
    )(p1, a1, Wq_y, Wk_y, Wv_y, Wo_y, bo_y2, Wk_x, Wv_x)

    cblk = 256
    anblk = n // cblk
    ablk_spec = pl.BlockSpec((1, cblk, d), lambda i, j: (i, j, 0))
    lvl_specs = []
    lvl_args_k = []
    lvl_args_v = []
    rest_boff = {3: 0, 4: 16, 5: 48}
    for l in range(1, 6):
        m = cblk >> l
        if l == 1:
            src_k, src_v = kp0, vp0
            boff = 0
        elif l == 2:
            src_k, src_v = kp1, vp1
            boff = 0
        else:
            src_k, src_v = kpr, vpr
            boff = rest_boff[l]
        lvl_specs.append(
            pl.BlockSpec((1, m, d), lambda i, j, boff=boff: (i, boff + j, 0)))
        lvl_args_k.append(src_k)
        lvl_args_v.append(src_v)
    tail_spec = pl.BlockSpec((1, 64, d), lambda i, j: (i, 7, 0))

    out = pl.pallas_call(
        lambda xr, wq, wk, wv, k1, k2, k3, k4, k5, kt,
        v1, v2, v3, v4, v5, vt, wor, bor, orf: _attn_kernel(
            xr, wq, wk, wv, [k1, k2, k3, k4, k5], [v1, v2, v3, v4, v5],
            kt, vt, wor, bor, orf),
        grid=(b, anblk),
        in_specs=[ablk_spec] + wspec(3) + lvl_specs + [tail_spec]
        + lvl_specs + [tail_spec] + wspec(1) + [bspec],
        out_specs=ablk_spec,
        out_shape=jax.ShapeDtypeStruct((b, n, d), f32),
    )(x, Wq_x, Wk_x, Wv_x, *lvl_args_k, kpr, *lvl_args_v, vpr, Wo_x, bo_x2)
    return out


# speedup vs baseline: 2.0757x; 1.0066x over previous
import math

import jax
import jax.numpy as jnp
from jax.experimental import pallas as pl

_H = 16


def _mm(a, w):
    return jax.lax.dot_general(
        a, w, (((1,), (1,)), ((), ())), preferred_element_type=jnp.float32
    )


def _dot(a, b):
    return jax.lax.dot_general(
        a, b, (((1,), (0,)), ((), ())), preferred_element_type=jnp.float32
    )


def _iot(shape, dim):
    return jax.lax.broadcasted_iota(jnp.int32, shape, dim)


def _seg_mats(d, h):
    dh = d // h
    seg = jnp.where(_iot((d, h), 0) // dh == _iot((d, h), 1), 1.0, 0.0)
    exp_m = jnp.where(_iot((h, d), 0) == _iot((h, d), 1) // dh, 1.0, 0.0)
    return seg, exp_m


def _pairavg(a, pc, d):
    ar = a.reshape(pc, 2, d)
    return 0.5 * (ar[:, 0, :] + ar[:, 1, :])


def _tree_step(children, avg_c, wq, wkv, scale, h):
    pc, d = avg_c.shape
    seg, exp_m = _seg_mats(d, h)
    q = _mm(avg_c, wq)
    kv = _mm(children, wkv)
    kr = kv[:, :d].reshape(pc, 2, d)
    vr = kv[:, d:].reshape(pc, 2, d)
    s0 = _dot(q * kr[:, 0, :], seg)
    s1 = _dot(q * kr[:, 1, :], seg)
    w0 = jax.nn.sigmoid((s0 - s1) * scale)
    w0f = _dot(w0, exp_m)
    return w0f * vr[:, 0, :] + (1.0 - w0f) * vr[:, 1, :]


def _kv_of_parents(parents, wo, bo, wkvx):
    y = _mm(parents, wo) + bo
    kv = _mm(y, wkvx)
    d = parents.shape[1]
    return kv[:, :d], kv[:, d:]


def _l0_kernel(x_ref, wq, wkv, wo, bo, wkvx, par_ref, avg_ref, kp_ref, vp_ref):
    nb, n2, d = x_ref.shape
    xb = x_ref[...].reshape(nb * n2, d)
    h = _H
    scale = 1.0 / math.sqrt(d // h)
    pc = nb * n2 // 2
    avg_c = _pairavg(xb, pc, d)
    parents = _tree_step(xb, avg_c, wq[...], wkv[...], scale, h)
    par_ref[...] = parents.reshape(nb, n2 // 2, d)
    avg_ref[...] = avg_c.reshape(nb, n2 // 2, d)
    kp, vp = _kv_of_parents(parents, wo[...], bo[...], wkvx[...])
    kp_ref[...] = kp.reshape(nb, n2 // 2, d)
    vp_ref[...] = vp.reshape(nb, n2 // 2, d)


def _l1_kernel(ch_ref, avgp_ref, wq, wkv, wo, bo, wkvx,
               par_ref, avg_ref, kp_ref, vp_ref):
    nb, n2, d = ch_ref.shape
    ch = ch_ref[...].reshape(nb * n2, d)
    h = _H
    scale = 1.0 / math.sqrt(d // h)
    pc = nb * n2 // 2
    avg_c = _pairavg(avgp_ref[...].reshape(nb * n2, d), pc, d)
    parents = _tree_step(ch, avg_c, wq[...], wkv[...], scale, h)
    par_ref[...] = parents.reshape(nb, n2 // 2, d)
    avg_ref[...] = avg_c.reshape(nb, n2 // 2, d)
    kp, vp = _kv_of_parents(parents, wo[...], bo[...], wkvx[...])
    kp_ref[...] = kp.reshape(nb, n2 // 2, d)
    vp_ref[...] = vp.reshape(nb, n2 // 2, d)


def _rest_kernel(ch_ref, avgp_ref, wq, wkv, wo, bo, wkvx, kp_ref, vp_ref):
    nb, rows, d = ch_ref.shape
    h = _H
    scale = 1.0 / math.sqrt(d // h)
    children = ch_ref[...].reshape(nb * rows, d)
    avg_prev = avgp_ref[...].reshape(nb * rows, d)
    roff = 0
    pc = rows // 2
    while pc >= 1:
        avg_c = _pairavg(avg_prev, nb * pc, d)
        parents = _tree_step(children, avg_c, wq[...], wkv[...], scale, h)
        kp, vp = _kv_of_parents(parents, wo[...], bo[...], wkvx[...])
        kp_ref[:, roff:roff + pc, :] = kp.reshape(nb, pc, d)
        vp_ref[:, roff:roff + pc, :] = vp.reshape(nb, pc, d)
        children = parents
        avg_prev = avg_c
        roff += pc
        pc //= 2
    kp_ref[:, roff:roff + 1, :] = jnp.zeros((nb, 1, d), jnp.float32)
    vp_ref[:, roff:roff + 1, :] = jnp.zeros((nb, 1, d), jnp.float32)


def _attn_kernel(x_ref, wqkv_ref, ks, vs, ktail_ref, vtail_ref,
                 wo_ref, bo_ref, o_ref):
    xc = x_ref[0]
    c, d = xc.shape
    qkv = _mm(xc, wqkv_ref[...])
    q = qkv[:, :d]
    kleaf = qkv[:, d:2 * d]
    vleaf = qkv[:, 2 * d:]
    h = _H
    scale = 1.0 / math.sqrt(d // h)
    j = pl.program_id(1)
    c0 = j * c
    L = 11
    f32 = jnp.float32
    seg, exp_m = _seg_mats(d, h)

    sels = []
    for l in range(L):
        if l == 0:
            m = c
            t = _iot((c, m), 0) ^ 1
        elif (c >> l) >= 8:
            m = c >> l
            t = (_iot((c, m), 0) >> l) ^ 1
        else:
            m = 64
            loc_off = {6: 0, 7: 32, 8: 48, 9: 56, 10: 60}[l]
            t = (((c0 + _iot((c, m), 0)) >> l) ^ 1) + loc_off
        sels.append(jnp.where(_iot((c, m), 1) == t, 1.0, 0.0))

    def level_rows(l, leaf_val, lvl_refs, tail_ref):
        if l == 0:
            src = leaf_val
        elif (c >> l) >= 8:
            src = lvl_refs[l - 1][0]
        else:
            src = tail_ref[0]
        return _dot(sels[l], src)

    scores = []
    for l in range(L):
        ke = level_rows(l, kleaf, ks, ktail_ref)
        scores.append(_dot(q * ke, seg) * scale)
    mx = scores[0]
    for s in scores[1:]:
        mx = jnp.maximum(mx, s)
    exps = [jnp.exp(s - mx) for s in scores]
    tot = exps[0]
    for e in exps[1:]:
        tot = tot + e
    inv = 1.0 / tot
    acc = jnp.zeros((c, d), f32)
    for l in range(L):
        wf = _dot(exps[l] * inv, exp_m)
        ve = level_rows(l, vleaf, vs, vtail_ref)
        acc = acc + wf * ve
    o_ref[0] = _mm(acc, wo_ref[...]) + bo_ref[...]


def kernel(x, Wq_y, Wk_y, Wv_y, Wo_y, bo_y, Wq_x, Wk_x, Wv_x, Wo_x, bo_x):
    b, n, d = x.shape
    f32 = jnp.float32
    bo_y2 = bo_y.reshape(1, d)
    bo_x2 = bo_x.reshape(1, d)
    wkv_y = jnp.concatenate([Wk_y, Wv_y], axis=0)
    wkv_x = jnp.concatenate([Wk_x, Wv_x], axis=0)
    wqkv_x = jnp.concatenate([Wq_x, Wk_x, Wv_x], axis=0)

    def cspec(rows):
        return pl.BlockSpec((rows, d), lambda *a: (0, 0))

    bspec = pl.BlockSpec((1, d), lambda *a: (0, 0))

    cp0 = 256
    g0 = (n // 2) // cp0
    p0, a0, kp0, vp0 = pl.pallas_call(
        _l0_kernel,
        grid=(g0,),
        in_specs=[pl.BlockSpec((b, 2 * cp0, d), lambda j: (0, j, 0)),
                  cspec(d), cspec(2 * d), cspec(d), bspec, cspec(2 * d)],
        out_specs=[pl.BlockSpec((b, cp0, d), lambda j: (0, j, 0))] * 4,
        out_shape=[jax.ShapeDtypeStruct((b, n // 2, d), f32)] * 4,
    )(x, Wq_y, wkv_y, Wo_y, bo_y2, wkv_x)

    cp1 = 128
    g1 = (n // 4) // cp1
    p1, a1, kp1, vp1 = pl.pallas_call(
        _l1_kernel,
        grid=(g1,),
        in_specs=[pl.BlockSpec((b, 2 * cp1, d), lambda j: (0, j, 0))] * 2
        + [cspec(d), cspec(2 * d), cspec(d), bspec, cspec(2 * d)],
        out_specs=[pl.BlockSpec((b, cp1, d), lambda j: (0, j, 0))] * 4,
        out_shape=[jax.ShapeDtypeStruct((b, n // 4, d), f32)] * 4,
    )(p0, a0, Wq_y, wkv_y, Wo_y, bo_y2, wkv_x)

    r = n // 4
    kpr, vpr = pl.pallas_call(
        _rest_kernel,
        grid=(1,),
        in_specs=[pl.BlockSpec((b, r, d), lambda j: (0, 0, 0))] * 2
        + [cspec(d), cspec(2 * d), cspec(d), bspec, cspec(2 * d)],
        out_specs=[pl.BlockSpec((b, r, d), lambda j: (0, 0, 0))] * 2,
        out_shape=[jax.ShapeDtypeStruct((b, r, d), f32)] * 2,
    )(p1, a1, Wq_y, wkv_y, Wo_y, bo_y2, wkv_x)

    cblk = 256
    anblk = n // cblk
    ablk_spec = pl.BlockSpec((1, cblk, d), lambda i, j: (i, j, 0))
    lvl_specs = []
    lvl_args_k = []
    lvl_args_v = []
    rest_boff = {3: 0, 4: 16, 5: 48}
    for l in range(1, 6):
        m = cblk >> l
        if l == 1:
            src_k, src_v = kp0, vp0
            boff = 0
        elif l == 2:
            src_k, src_v = kp1, vp1
            boff = 0
        else:
            src_k, src_v = kpr, vpr
            boff = rest_boff[l]
        lvl_specs.append(
            pl.BlockSpec((1, m, d), lambda i, j, boff=boff: (i, boff + j, 0)))
        lvl_args_k.append(src_k)
        lvl_args_v.append(src_v)
    tail_spec = pl.BlockSpec((1, 64, d), lambda i, j: (i, 7, 0))

    out = pl.pallas_call(
        lambda xr, wqkv, k1, k2, k3, k4, k5, kt,
        v1, v2, v3, v4, v5, vt, wor, bor, orf: _attn_kernel(
            xr, wqkv, [k1, k2, k3, k4, k5], [v1, v2, v3, v4, v5],
            kt, vt, wor, bor, orf),
        grid=(b, anblk),
        in_specs=[ablk_spec, pl.BlockSpec((3 * d, d), lambda i, j: (0, 0))]
        + lvl_specs + [tail_spec] + lvl_specs + [tail_spec]
        + [pl.BlockSpec((d, d), lambda i, j: (0, 0)),
           pl.BlockSpec((1, d), lambda i, j: (0, 0))],
        out_specs=ablk_spec,
        out_shape=jax.ShapeDtypeStruct((b, n, d), f32),
    )(x, wqkv_x, *lvl_args_k, kpr, *lvl_args_v, vpr, Wo_x, bo_x2)
    return out
